# baseline (speedup 1.0000x reference)
import functools

import jax
import jax.numpy as jnp
from jax import lax
from jax.experimental import pallas as pl
from jax.experimental.pallas import tpu as pltpu
from jax.experimental.pallas import tpu_sc as plsc

NPOS = 1000
EMB_DIM = 64
BATCH = 4096
HIST = 200

NC = 2
NS = 16
NW = NC * NS

B = BATCH * HIST
B_PER_W = B // NW
BATCH_PER_W = BATCH // NW

SPLITS = ((0, 104), (104, 96))
N_HALF = 2 * BATCH_PER_W
NBUF = 4

_mesh = plsc.VectorSubcoreMesh(core_axis_name="c", subcore_axis_name="s")


@functools.partial(
    pl.kernel,
    out_type=jax.ShapeDtypeStruct((BATCH, HIST, EMB_DIM), jnp.float32),
    mesh=_mesh,
    compiler_params=pltpu.CompilerParams(use_tc_tiling_on_sc=True),
    scratch_types=[
        pltpu.VMEM((B_PER_W,), jnp.int32),
        pltpu.VMEM((NBUF, 104, EMB_DIM), jnp.float32),
        pltpu.VMEM_SHARED((NPOS, EMB_DIM), jnp.float32),
    ] + [pltpu.SemaphoreType.DMA] * (2 * NBUF + 1),
)
def _emb_lookup(x_hbm, w_hbm, out_hbm, idx_all, rows_v, tab_sh, *sems):
    gsem = sems[:NBUF]
    ssem = sems[NBUF:2 * NBUF]
    lsem = sems[2 * NBUF]

    wid = lax.axis_index("s") * NC + lax.axis_index("c")
    base = wid * B_PER_W
    b0 = wid * BATCH_PER_W

    idx_cp = pltpu.async_copy(
        x_hbm.at[pl.ds(base, B_PER_W)], idx_all, lsem)

    @pl.when(lax.axis_index("s") == 0)
    def _():
        pltpu.sync_copy(w_hbm, tab_sh)

    idx_cp.wait()
    plsc.subcore_barrier()

    def issue_gather(b, s, off, n):
        pltpu.async_copy(
            tab_sh.at[idx_all.at[pl.ds(b * HIST + off, n)]],
            rows_v.at[s, pl.ds(0, n)],
            gsem[s])

    def wait_gather(b, s, off, n):
        pltpu.make_async_copy(
            tab_sh.at[idx_all.at[pl.ds(b * HIST + off, n)]],
            rows_v.at[s, pl.ds(0, n)],
            gsem[s]).wait()

    def issue_store(b, s, off, n):
        pltpu.async_copy(
            rows_v.at[s, pl.ds(0, n)],
            out_hbm.at[b0 + b, pl.ds(off, n)], ssem[s])

    def wait_store(b, s, off, n):
        pltpu.make_async_copy(
            rows_v.at[s, pl.ds(0, n)],
            out_hbm.at[b0 + b, pl.ds(off, n)], ssem[s]).wait()

    def half(h):
        return h // 2, SPLITS[h % 2]

    for s in range(NBUF):
        b, (off, n) = half(s)
        issue_gather(b, s, off, n)

    @pl.loop(0, N_HALF, step=NBUF)
    def _(g):
        gb = g // 2
        for s in range(NBUF):
            b = gb + s // 2
            off, n = SPLITS[s % 2]
            wait_gather(b, s, off, n)
            issue_store(b, s, off, n)

            @pl.when(g + s + NBUF < N_HALF)
            def _():
                wait_store(b, s, off, n)
                issue_gather(b + NBUF // 2, s, off, n)

    for s in range(NBUF):
        b, (off, n) = half(N_HALF - NBUF + s)
        wait_store(b, s, off, n)


def kernel(x, pos_emb_weight):
    w = pos_emb_weight.at[0].set(0.0)
    x_flat = x.astype(jnp.int32).reshape(B)
    return _emb_lookup(x_flat, w)


# speedup vs baseline: 1.5720x; 1.5720x over previous
import functools

import jax
import jax.numpy as jnp
from jax import lax
from jax.experimental import pallas as pl
from jax.experimental.pallas import tpu as pltpu
from jax.experimental.pallas import tpu_sc as plsc

NPOS = 1000
EMB_DIM = 64
BATCH = 4096
HIST = 200

NC = 2
NS = 16
NW = NC * NS

B = BATCH * HIST
B_PER_W = B // NW

IDXW = 128
CHUNK_ROWS = 1
CHUNK = CHUNK_ROWS * IDXW
N_CHUNKS = B_PER_W // CHUNK
ROWS_PER_W = B_PER_W // IDXW
NBUF = 8

_mesh = plsc.VectorSubcoreMesh(core_axis_name="c", subcore_axis_name="s")


@functools.partial(
    pl.kernel,
    out_type=jax.ShapeDtypeStruct((B, 2 * EMB_DIM), jnp.float32),
    mesh=_mesh,
    compiler_params=pltpu.CompilerParams(use_tc_tiling_on_sc=False),
    scratch_types=[
        pltpu.VMEM((ROWS_PER_W, IDXW), jnp.int32),
        pltpu.VMEM((NBUF, CHUNK, EMB_DIM), jnp.float32),
        pltpu.VMEM_SHARED((NPOS, EMB_DIM), jnp.float32),
    ] + [pltpu.SemaphoreType.DMA] * 17,
)
def _emb_lookup(x_hbm, w_hbm, out_hbm, idx_all, rows_v, tab_sh, *sems):
    gsem = sems[:NBUF]
    ssem = sems[NBUF:2 * NBUF]
    lsem = sems[2 * NBUF]

    wid = lax.axis_index("s") * NC + lax.axis_index("c")
    base = wid * B_PER_W
    brow = wid * ROWS_PER_W

    idx_cp = pltpu.async_copy(
        x_hbm.at[pl.ds(brow, ROWS_PER_W)], idx_all, lsem)

    @pl.when(lax.axis_index("s") == 0)
    def _():
        pltpu.sync_copy(w_hbm, tab_sh)

    idx_cp.wait()
    plsc.subcore_barrier()

    def issue_gathers(c, s):
        for j in range(CHUNK_ROWS):
            pltpu.async_copy(
                tab_sh.at[idx_all.at[c * CHUNK_ROWS + j]],
                rows_v.at[s, pl.ds(j * IDXW, IDXW)],
                gsem[s])

    def wait_gathers(c, s):
        for j in range(CHUNK_ROWS):
            pltpu.make_async_copy(
                tab_sh.at[idx_all.at[c * CHUNK_ROWS + j]],
                rows_v.at[s, pl.ds(j * IDXW, IDXW)],
                gsem[s]).wait()

    def issue_store(c, s):
        pltpu.async_copy(
            rows_v.at[s],
            out_hbm.at[pl.ds(base + c * CHUNK, CHUNK), pl.ds(0, EMB_DIM)],
            ssem[s])

    def wait_store(c, s):
        pltpu.make_async_copy(
            rows_v.at[s],
            out_hbm.at[pl.ds(base + c * CHUNK, CHUNK), pl.ds(0, EMB_DIM)],
            ssem[s]).wait()

    for s in range(NBUF):
        issue_gathers(s, s)

    @pl.loop(0, N_CHUNKS, step=NBUF)
    def _(g):
        for s in range(NBUF):
            c = g + s
            wait_gathers(c, s)
            issue_store(c, s)

            @pl.when(c + NBUF < N_CHUNKS)
            def _():
                wait_store(c, s)
                issue_gathers(c + NBUF, s)

    for s in range(NBUF):
        wait_store(N_CHUNKS - NBUF + s, s)


def kernel(x, pos_emb_weight):
    w = pos_emb_weight.at[0].set(0.0)
    x_rows = x.astype(jnp.int32).reshape(B // IDXW, IDXW)
    out = _emb_lookup(x_rows, w)
    return out[:, :EMB_DIM].reshape(BATCH, HIST, EMB_DIM)
